# Initial kernel scaffold; baseline (speedup 1.0000x reference)
import functools

import jax
import jax.numpy as jnp
from jax import lax
from jax.experimental import pallas as pl
from jax.experimental.pallas import tpu as pltpu
from jax.experimental.pallas import tpu_sc as plsc

_N = 10000
_E = 320000
_NP = 10240
_CH = 128
_NW = 32
_KD = 80
_EP = _NW * _KD * _CH
_RPS = _NP // 16
_BLK = 512
_GRID = _NP // _BLK

_MESH = plsc.VectorSubcoreMesh(core_axis_name="c", subcore_axis_name="s")


@functools.partial(
    pl.kernel,
    out_type=jax.ShapeDtypeStruct((2, _NP, 16), jnp.float32),
    mesh=_MESH,
    scratch_types=[
        pltpu.VMEM((_KD, _CH), jnp.int32),
        pltpu.VMEM((_CH, 16), jnp.float32),
        pltpu.VMEM((_RPS, 16), jnp.float32),
        pltpu.VMEM_SHARED((_NP, 16), jnp.float32),
    ],
)
def _sc_degree(dst_hbm, onehot_hbm, zeros_hbm, out_hbm, dst_v, one_v, z_v, acc_sh):
    c = lax.axis_index("c")
    s = lax.axis_index("s")
    wid = s * 2 + c
    r0 = s * _RPS
    pltpu.sync_copy(zeros_hbm, z_v)
    pltpu.sync_copy(z_v, acc_sh.at[pl.ds(r0, _RPS)])
    pltpu.sync_copy(onehot_hbm, one_v)
    pltpu.sync_copy(dst_hbm.at[wid], dst_v)
    plsc.subcore_barrier()

    def body(j, carry):
        pltpu.sync_copy(one_v, acc_sh.at[dst_v.at[j]], add=True)
        return carry

    lax.fori_loop(0, _KD, body, 0)
    plsc.subcore_barrier()
    pltpu.sync_copy(acc_sh.at[pl.ds(r0, _RPS)], out_hbm.at[c, pl.ds(r0, _RPS)])


@functools.partial(
    pl.kernel,
    out_type=jax.ShapeDtypeStruct((2, _NP, 128), jnp.float32),
    mesh=_MESH,
    scratch_types=[
        pltpu.VMEM((_KD, _CH), jnp.int32),
        pltpu.VMEM((_KD, _CH), jnp.int32),
        pltpu.VMEM((_CH, 128), jnp.float32),
        pltpu.VMEM((_CH, 128), jnp.float32),
        pltpu.VMEM_SHARED((_NP, 128), jnp.float32),
        pltpu.SemaphoreType.DMA,
        pltpu.SemaphoreType.DMA,
    ],
)
def _sc_agg(y_hbm, src_hbm, dst_hbm, zeros_hbm, out_hbm,
            src_v, dst_v, m0, m1, acc_sh, sem0, sem1):
    c = lax.axis_index("c")
    s = lax.axis_index("s")
    wid = s * 2 + c
    r0 = s * _RPS
    pltpu.sync_copy(zeros_hbm, m0)
    for i in range(_RPS // _CH):
        pltpu.sync_copy(m0, acc_sh.at[pl.ds(r0 + i * _CH, _CH)])
    pltpu.sync_copy(src_hbm.at[wid], src_v)
    pltpu.sync_copy(dst_hbm.at[wid], dst_v)
    plsc.subcore_barrier()

    pltpu.async_copy(y_hbm.at[src_v.at[0]], m0, sem0)

    def body(jj, carry):
        j = jj * 2
        pltpu.make_async_copy(y_hbm.at[src_v.at[j]], m0, sem0).wait()
        pltpu.async_copy(y_hbm.at[src_v.at[j + 1]], m1, sem1)
        pltpu.sync_copy(m0, acc_sh.at[dst_v.at[j]], add=True)
        pltpu.make_async_copy(y_hbm.at[src_v.at[j + 1]], m1, sem1).wait()

        @pl.when(jj + 1 < _KD // 2)
        def _():
            pltpu.async_copy(y_hbm.at[src_v.at[j + 2]], m0, sem0)

        pltpu.sync_copy(m1, acc_sh.at[dst_v.at[j + 1]], add=True)
        return carry

    lax.fori_loop(0, _KD // 2, body, 0)
    plsc.subcore_barrier()
    pltpu.sync_copy(acc_sh.at[pl.ds(r0, _RPS)], out_hbm.at[c, pl.ds(r0, _RPS)])


def _dinv(d0_ref, d1_ref):
    return lax.rsqrt(1.0 + d0_ref[:, 0:1] + d1_ref[:, 0:1])


def _mm_scale(x, w, d0, d1):
    def body(x_ref, w_ref, d0_ref, d1_ref, o_ref):
        dinv = _dinv(d0_ref, d1_ref)
        o_ref[...] = jnp.dot(x_ref[...], w_ref[...],
                             preferred_element_type=jnp.float32) * dinv

    return pl.pallas_call(
        body,
        grid=(_GRID,),
        in_specs=[
            pl.BlockSpec((_BLK, 128), lambda i: (i, 0)),
            pl.BlockSpec((128, 128), lambda i: (0, 0)),
            pl.BlockSpec((_BLK, 16), lambda i: (i, 0)),
            pl.BlockSpec((_BLK, 16), lambda i: (i, 0)),
        ],
        out_specs=pl.BlockSpec((_BLK, 128), lambda i: (i, 0)),
        out_shape=jax.ShapeDtypeStruct((_NP, 128), jnp.float32),
    )(x, w, d0, d1)


def _layer2(p0, p1, y1, d0, d1, w2, b1r):
    def body(p0_ref, p1_ref, y_ref, d0_ref, d1_ref, w_ref, b_ref, o_ref):
        dinv = _dinv(d0_ref, d1_ref)
        h = jnp.maximum(b_ref[...] + dinv * (p0_ref[...] + p1_ref[...] + y_ref[...]), 0.0)
        o_ref[...] = jnp.dot(h, w_ref[...],
                             preferred_element_type=jnp.float32) * dinv

    return pl.pallas_call(
        body,
        grid=(_GRID,),
        in_specs=[
            pl.BlockSpec((_BLK, 128), lambda i: (i, 0)),
            pl.BlockSpec((_BLK, 128), lambda i: (i, 0)),
            pl.BlockSpec((_BLK, 128), lambda i: (i, 0)),
            pl.BlockSpec((_BLK, 16), lambda i: (i, 0)),
            pl.BlockSpec((_BLK, 16), lambda i: (i, 0)),
            pl.BlockSpec((128, 128), lambda i: (0, 0)),
            pl.BlockSpec((1, 128), lambda i: (0, 0)),
        ],
        out_specs=pl.BlockSpec((_BLK, 128), lambda i: (i, 0)),
        out_shape=jax.ShapeDtypeStruct((_NP, 128), jnp.float32),
    )(p0, p1, y1, d0, d1, w2, b1r)


def _final(q0, q1, y2, d0, d1, wh, b2r, bhr):
    def body(q0_ref, q1_ref, y_ref, d0_ref, d1_ref, w_ref, b2_ref, bh_ref, o_ref):
        dinv = _dinv(d0_ref, d1_ref)
        h = jnp.maximum(b2_ref[...] + dinv * (q0_ref[...] + q1_ref[...] + y_ref[...]), 0.0)
        o_ref[...] = jnp.dot(h, w_ref[...],
                             preferred_element_type=jnp.float32) + bh_ref[...]

    return pl.pallas_call(
        body,
        grid=(_GRID,),
        in_specs=[
            pl.BlockSpec((_BLK, 128), lambda i: (i, 0)),
            pl.BlockSpec((_BLK, 128), lambda i: (i, 0)),
            pl.BlockSpec((_BLK, 128), lambda i: (i, 0)),
            pl.BlockSpec((_BLK, 16), lambda i: (i, 0)),
            pl.BlockSpec((_BLK, 16), lambda i: (i, 0)),
            pl.BlockSpec((128, 16), lambda i: (0, 0)),
            pl.BlockSpec((1, 128), lambda i: (0, 0)),
            pl.BlockSpec((1, 16), lambda i: (0, 0)),
        ],
        out_specs=pl.BlockSpec((_BLK, 16), lambda i: (i, 0)),
        out_shape=jax.ShapeDtypeStruct((_NP, 16), jnp.float32),
    )(q0, q1, y2, d0, d1, wh, b2r, bhr)


def kernel(obs, edge_index, W1, b1, W2, b2, Wm, bm, Wl, bl):
    f32 = jnp.float32
    obs_p = jnp.zeros((_NP, obs.shape[1]), f32).at[:_N].set(obs)
    src = edge_index[0]
    dst = edge_index[1]
    padn = _EP - _E
    src_p = jnp.concatenate(
        [src, jnp.zeros((padn,), jnp.int32)]).reshape(_NW, _KD, _CH)
    dst_p = jnp.concatenate(
        [dst, jnp.full((padn,), _N, jnp.int32)]).reshape(_NW, _KD, _CH)
    onehot = jnp.zeros((_CH, 16), f32).at[:, 0].set(1.0)
    zeros16 = jnp.zeros((_RPS, 16), f32)
    zeros128 = jnp.zeros((_CH, 128), f32)

    degp = _sc_degree(dst_p, onehot, zeros16)
    d0, d1 = degp[0], degp[1]

    y1 = _mm_scale(obs_p, W1, d0, d1)
    p = _sc_agg(y1, src_p, dst_p, zeros128)
    y2 = _layer2(p[0], p[1], y1, d0, d1, W2, b1.reshape(1, -1))
    q = _sc_agg(y2, src_p, dst_p, zeros128)

    wh = jnp.concatenate([Wm, Wl], axis=1)
    bh = jnp.concatenate([bm, bl]).reshape(1, -1)
    heads = _final(q[0], q[1], y2, d0, d1, wh, b2.reshape(1, -1), bh)
    return heads[:_N, :8], heads[:_N, 8:]


# speedup vs baseline: 7.7546x; 7.7546x over previous
import functools

import jax
import jax.numpy as jnp
from jax import lax
from jax.experimental import pallas as pl
from jax.experimental.pallas import tpu as pltpu
from jax.experimental.pallas import tpu_sc as plsc

_N = 10000
_E = 320000
_NP = 10240
_CH = 128
_NW = 32
_KD = 80
_GI = 16
_EP = _NW * _KD * _CH
_RPS = _NP // 16
_BLK = 512
_GRID = _NP // _BLK

_MESH = plsc.VectorSubcoreMesh(core_axis_name="c", subcore_axis_name="s")


@functools.partial(
    pl.kernel,
    out_type=jax.ShapeDtypeStruct((2, 16, _RPS, 128), jnp.float32),
    mesh=_MESH,
    scratch_types=[
        pltpu.VMEM((_KD, _CH), jnp.int32),
        pltpu.VMEM((_CH, 128), jnp.float32),
        pltpu.VMEM_SHARED((_NP, 128), jnp.float32),
    ],
)
def _sc_degree(dst_hbm, onehot_hbm, zeros_hbm, out_hbm, dst_v, buf_v, acc_sh):
    c = lax.axis_index("c")
    s = lax.axis_index("s")
    wid = s * 2 + c
    r0 = s * _RPS
    pltpu.sync_copy(zeros_hbm, buf_v)
    for i in range(_RPS // _CH):
        pltpu.sync_copy(buf_v, acc_sh.at[pl.ds(r0 + i * _CH, _CH)])
    pltpu.sync_copy(onehot_hbm, buf_v)
    pltpu.sync_copy(dst_hbm.at[wid], dst_v)
    plsc.subcore_barrier()

    def body(j, carry):
        pltpu.sync_copy(buf_v, acc_sh.at[dst_v.at[j]], add=True)
        return carry

    lax.fori_loop(0, _KD, body, 0)
    plsc.subcore_barrier()
    for i in range(_RPS // _CH):
        pltpu.sync_copy(acc_sh.at[pl.ds(r0 + i * _CH, _CH)], buf_v)
        pltpu.sync_copy(buf_v, out_hbm.at[c, s, pl.ds(i * _CH, _CH)])


@functools.partial(
    pl.kernel,
    out_type=jax.ShapeDtypeStruct((2, 16, _RPS, 128), jnp.float32),
    mesh=_MESH,
    scratch_types=[
        pltpu.VMEM((_GI, _CH), jnp.int32),
        pltpu.VMEM((_GI, _CH), jnp.int32),
        pltpu.VMEM((_CH, 128), jnp.float32),
        pltpu.VMEM((_CH, 128), jnp.float32),
        pltpu.VMEM_SHARED((_NP, 128), jnp.float32),
        pltpu.SemaphoreType.DMA,
        pltpu.SemaphoreType.DMA,
    ],
)
def _sc_agg(y_hbm, src_hbm, dst_hbm, zeros_hbm, out_hbm,
            src_v, dst_v, m0, m1, acc_sh, sem0, sem1):
    c = lax.axis_index("c")
    s = lax.axis_index("s")
    wid = s * 2 + c
    r0 = s * _RPS
    pltpu.sync_copy(zeros_hbm, m0)
    for i in range(_RPS // _CH):
        pltpu.sync_copy(m0, acc_sh.at[pl.ds(r0 + i * _CH, _CH)])
    plsc.subcore_barrier()

    for g in range(_KD // _GI):
        pltpu.sync_copy(src_hbm.at[wid, pl.ds(g * _GI, _GI)], src_v)
        pltpu.sync_copy(dst_hbm.at[wid, pl.ds(g * _GI, _GI)], dst_v)
        pltpu.async_copy(y_hbm.at[src_v.at[0]], m0, sem0)

        def body(jj, carry):
            j = jj * 2
            pltpu.make_async_copy(y_hbm.at[src_v.at[j]], m0, sem0).wait()
            pltpu.async_copy(y_hbm.at[src_v.at[j + 1]], m1, sem1)
            pltpu.sync_copy(m0, acc_sh.at[dst_v.at[j]], add=True)
            pltpu.make_async_copy(y_hbm.at[src_v.at[j + 1]], m1, sem1).wait()

            @pl.when(jj + 1 < _GI // 2)
            def _():
                pltpu.async_copy(y_hbm.at[src_v.at[j + 2]], m0, sem0)

            pltpu.sync_copy(m1, acc_sh.at[dst_v.at[j + 1]], add=True)
            return carry

        lax.fori_loop(0, _GI // 2, body, 0)
    plsc.subcore_barrier()
    for i in range(_RPS // _CH):
        pltpu.sync_copy(acc_sh.at[pl.ds(r0 + i * _CH, _CH)], m0)
        pltpu.sync_copy(m0, out_hbm.at[c, s, pl.ds(i * _CH, _CH)])


def _dinv(d0_ref, d1_ref):
    return lax.rsqrt(1.0 + d0_ref[:, 0:1] + d1_ref[:, 0:1])


def _mm_scale(x, w, d0, d1):
    def body(x_ref, w_ref, d0_ref, d1_ref, o_ref):
        dinv = _dinv(d0_ref, d1_ref)
        o_ref[...] = jnp.dot(x_ref[...], w_ref[...],
                             preferred_element_type=jnp.float32) * dinv

    return pl.pallas_call(
        body,
        grid=(_GRID,),
        in_specs=[
            pl.BlockSpec((_BLK, 128), lambda i: (i, 0)),
            pl.BlockSpec((128, 128), lambda i: (0, 0)),
            pl.BlockSpec((_BLK, 128), lambda i: (i, 0)),
            pl.BlockSpec((_BLK, 128), lambda i: (i, 0)),
        ],
        out_specs=pl.BlockSpec((_BLK, 128), lambda i: (i, 0)),
        out_shape=jax.ShapeDtypeStruct((_NP, 128), jnp.float32),
    )(x, w, d0, d1)


def _layer2(p0, p1, y1, d0, d1, w2, b1r):
    def body(p0_ref, p1_ref, y_ref, d0_ref, d1_ref, w_ref, b_ref, o_ref):
        dinv = _dinv(d0_ref, d1_ref)
        h = jnp.maximum(b_ref[...] + dinv * (p0_ref[...] + p1_ref[...] + y_ref[...]), 0.0)
        o_ref[...] = jnp.dot(h, w_ref[...],
                             preferred_element_type=jnp.float32) * dinv

    return pl.pallas_call(
        body,
        grid=(_GRID,),
        in_specs=[
            pl.BlockSpec((_BLK, 128), lambda i: (i, 0)),
            pl.BlockSpec((_BLK, 128), lambda i: (i, 0)),
            pl.BlockSpec((_BLK, 128), lambda i: (i, 0)),
            pl.BlockSpec((_BLK, 128), lambda i: (i, 0)),
            pl.BlockSpec((_BLK, 128), lambda i: (i, 0)),
            pl.BlockSpec((128, 128), lambda i: (0, 0)),
            pl.BlockSpec((1, 128), lambda i: (0, 0)),
        ],
        out_specs=pl.BlockSpec((_BLK, 128), lambda i: (i, 0)),
        out_shape=jax.ShapeDtypeStruct((_NP, 128), jnp.float32),
    )(p0, p1, y1, d0, d1, w2, b1r)


def _final(q0, q1, y2, d0, d1, wh, b2r, bhr):
    def body(q0_ref, q1_ref, y_ref, d0_ref, d1_ref, w_ref, b2_ref, bh_ref, o_ref):
        dinv = _dinv(d0_ref, d1_ref)
        h = jnp.maximum(b2_ref[...] + dinv * (q0_ref[...] + q1_ref[...] + y_ref[...]), 0.0)
        o_ref[...] = jnp.dot(h, w_ref[...],
                             preferred_element_type=jnp.float32) + bh_ref[...]

    return pl.pallas_call(
        body,
        grid=(_GRID,),
        in_specs=[
            pl.BlockSpec((_BLK, 128), lambda i: (i, 0)),
            pl.BlockSpec((_BLK, 128), lambda i: (i, 0)),
            pl.BlockSpec((_BLK, 128), lambda i: (i, 0)),
            pl.BlockSpec((_BLK, 128), lambda i: (i, 0)),
            pl.BlockSpec((_BLK, 128), lambda i: (i, 0)),
            pl.BlockSpec((128, 16), lambda i: (0, 0)),
            pl.BlockSpec((1, 128), lambda i: (0, 0)),
            pl.BlockSpec((1, 16), lambda i: (0, 0)),
        ],
        out_specs=pl.BlockSpec((_BLK, 16), lambda i: (i, 0)),
        out_shape=jax.ShapeDtypeStruct((_NP, 16), jnp.float32),
    )(q0, q1, y2, d0, d1, wh, b2r, bhr)


def kernel(obs, edge_index, W1, b1, W2, b2, Wm, bm, Wl, bl):
    f32 = jnp.float32
    obs_p = jnp.zeros((_NP, obs.shape[1]), f32).at[:_N].set(obs)
    src = edge_index[0]
    dst = edge_index[1]
    padn = _EP - _E
    src_p = jnp.concatenate(
        [src, jnp.zeros((padn,), jnp.int32)]).reshape(_NW, _KD, _CH)
    dst_p = jnp.concatenate(
        [dst, jnp.full((padn,), _N, jnp.int32)]).reshape(_NW, _KD, _CH)
    onehot = jnp.zeros((_CH, 128), f32).at[:, 0].set(1.0)
    zeros128 = jnp.zeros((_CH, 128), f32)

    degp = _sc_degree(dst_p, onehot, zeros128).reshape(2, _NP, 128)
    d0, d1 = degp[0], degp[1]

    y1 = _mm_scale(obs_p, W1, d0, d1)
    p = _sc_agg(y1, src_p, dst_p, zeros128).reshape(2, _NP, 128)
    y2 = _layer2(p[0], p[1], y1, d0, d1, W2, b1.reshape(1, -1))
    q = _sc_agg(y2, src_p, dst_p, zeros128).reshape(2, _NP, 128)

    wh = jnp.concatenate([Wm, Wl], axis=1)
    bh = jnp.concatenate([bm, bl]).reshape(1, -1)
    heads = _final(q[0], q[1], y2, d0, d1, wh, b2.reshape(1, -1), bh)
    return heads[:_N, :8], heads[:_N, 8:]


# speedup vs baseline: 9.2516x; 1.1930x over previous
import functools

import jax
import jax.numpy as jnp
from jax import lax
from jax.experimental import pallas as pl
from jax.experimental.pallas import tpu as pltpu
from jax.experimental.pallas import tpu_sc as plsc

_N = 10000
_E = 320000
_NP = 10240
_CH = 128
_NW = 32
_KD = 80
_GI = 16
_KD0 = 128
_KD1 = 32
_EP = _NW * _KD * _CH
_RPS = _NP // 16
_BLK = 512
_GRID = _NP // _BLK

_MESH = plsc.VectorSubcoreMesh(core_axis_name="c", subcore_axis_name="s")


@functools.partial(
    pl.kernel,
    out_type=jax.ShapeDtypeStruct((2, 16, _RPS, 128), jnp.float32),
    mesh=_MESH,
    scratch_types=[
        pltpu.VMEM((_KD, _CH), jnp.int32),
        pltpu.VMEM((_CH, 128), jnp.float32),
        pltpu.VMEM_SHARED((_NP, 128), jnp.float32),
    ],
)
def _sc_degree(dst_hbm, onehot_hbm, zeros_hbm, out_hbm, dst_v, buf_v, acc_sh):
    c = lax.axis_index("c")
    s = lax.axis_index("s")
    wid = s * 2 + c
    r0 = s * _RPS
    pltpu.sync_copy(zeros_hbm, buf_v)
    for i in range(_RPS // _CH):
        pltpu.sync_copy(buf_v, acc_sh.at[pl.ds(r0 + i * _CH, _CH)])
    pltpu.sync_copy(onehot_hbm, buf_v)
    pltpu.sync_copy(dst_hbm.at[wid], dst_v)
    plsc.subcore_barrier()

    def body(j, carry):
        pltpu.sync_copy(buf_v, acc_sh.at[dst_v.at[j]], add=True)
        return carry

    lax.fori_loop(0, _KD, body, 0)
    plsc.subcore_barrier()
    for i in range(_RPS // _CH):
        pltpu.sync_copy(acc_sh.at[pl.ds(r0 + i * _CH, _CH)], buf_v)
        pltpu.sync_copy(buf_v, out_hbm.at[c, s, pl.ds(i * _CH, _CH)])


@functools.partial(
    pl.kernel,
    out_type=jax.ShapeDtypeStruct((2, 16, _RPS, 128), jnp.float32),
    mesh=_MESH,
    scratch_types=[
        pltpu.VMEM((_GI, _CH), jnp.int32),
        pltpu.VMEM((_GI, _CH), jnp.int32),
        pltpu.VMEM((_CH, 128), jnp.float32),
        pltpu.VMEM((_CH, 128), jnp.float32),
        pltpu.VMEM_SHARED((_NP, 128), jnp.float32),
        pltpu.SemaphoreType.DMA,
        pltpu.SemaphoreType.DMA,
    ],
)
def _sc_agg(y_hbm, src_hbm, dst_hbm, zeros_hbm, out_hbm,
            src_v, dst_v, m0, m1, acc_sh, sem0, sem1):
    c = lax.axis_index("c")
    s = lax.axis_index("s")
    r0 = s * _RPS
    pltpu.sync_copy(zeros_hbm, m0)
    for i in range(_RPS // _CH):
        pltpu.sync_copy(m0, acc_sh.at[pl.ds(r0 + i * _CH, _CH)])
    plsc.subcore_barrier()

    def do_edges(base, nkd):
        for g in range(nkd // _GI):
            pltpu.sync_copy(src_hbm.at[pl.ds(base + g * _GI, _GI)], src_v)
            pltpu.sync_copy(dst_hbm.at[pl.ds(base + g * _GI, _GI)], dst_v)
            pltpu.async_copy(y_hbm.at[src_v.at[0]], m0, sem0)

            def body(jj, carry):
                j = jj * 2
                pltpu.make_async_copy(y_hbm.at[src_v.at[j]], m0, sem0).wait()
                pltpu.async_copy(y_hbm.at[src_v.at[j + 1]], m1, sem1)
                pltpu.sync_copy(m0, acc_sh.at[dst_v.at[j]], add=True)
                pltpu.make_async_copy(y_hbm.at[src_v.at[j + 1]], m1, sem1).wait()

                @pl.when(jj + 1 < _GI // 2)
                def _():
                    pltpu.async_copy(y_hbm.at[src_v.at[j + 2]], m0, sem0)

                pltpu.sync_copy(m1, acc_sh.at[dst_v.at[j + 1]], add=True)
                return carry

            lax.fori_loop(0, _GI // 2, body, 0)

    @pl.when(c == 0)
    def _():
        do_edges(s * _KD0, _KD0)

    @pl.when(c == 1)
    def _():
        do_edges(16 * _KD0 + s * _KD1, _KD1)

    plsc.subcore_barrier()
    for i in range(_RPS // _CH):
        pltpu.sync_copy(acc_sh.at[pl.ds(r0 + i * _CH, _CH)], m0)
        pltpu.sync_copy(m0, out_hbm.at[c, s, pl.ds(i * _CH, _CH)])


def _dinv(d0_ref, d1_ref):
    return lax.rsqrt(1.0 + d0_ref[:, 0:1] + d1_ref[:, 0:1])


def _mm_scale(x, w, d0, d1):
    def body(x_ref, w_ref, d0_ref, d1_ref, o_ref):
        dinv = _dinv(d0_ref, d1_ref)
        o_ref[...] = jnp.dot(x_ref[...], w_ref[...],
                             preferred_element_type=jnp.float32) * dinv

    return pl.pallas_call(
        body,
        grid=(_GRID,),
        in_specs=[
            pl.BlockSpec((_BLK, 128), lambda i: (i, 0)),
            pl.BlockSpec((128, 128), lambda i: (0, 0)),
            pl.BlockSpec((_BLK, 128), lambda i: (i, 0)),
            pl.BlockSpec((_BLK, 128), lambda i: (i, 0)),
        ],
        out_specs=pl.BlockSpec((_BLK, 128), lambda i: (i, 0)),
        out_shape=jax.ShapeDtypeStruct((_NP, 128), jnp.float32),
    )(x, w, d0, d1)


def _layer2(p0, p1, y1, d0, d1, w2, b1r):
    def body(p0_ref, p1_ref, y_ref, d0_ref, d1_ref, w_ref, b_ref, o_ref):
        dinv = _dinv(d0_ref, d1_ref)
        h = jnp.maximum(b_ref[...] + dinv * (p0_ref[...] + p1_ref[...] + y_ref[...]), 0.0)
        o_ref[...] = jnp.dot(h, w_ref[...],
                             preferred_element_type=jnp.float32) * dinv

    return pl.pallas_call(
        body,
        grid=(_GRID,),
        in_specs=[
            pl.BlockSpec((_BLK, 128), lambda i: (i, 0)),
            pl.BlockSpec((_BLK, 128), lambda i: (i, 0)),
            pl.BlockSpec((_BLK, 128), lambda i: (i, 0)),
            pl.BlockSpec((_BLK, 128), lambda i: (i, 0)),
            pl.BlockSpec((_BLK, 128), lambda i: (i, 0)),
            pl.BlockSpec((128, 128), lambda i: (0, 0)),
            pl.BlockSpec((1, 128), lambda i: (0, 0)),
        ],
        out_specs=pl.BlockSpec((_BLK, 128), lambda i: (i, 0)),
        out_shape=jax.ShapeDtypeStruct((_NP, 128), jnp.float32),
    )(p0, p1, y1, d0, d1, w2, b1r)


def _final(q0, q1, y2, d0, d1, wh, b2r, bhr):
    def body(q0_ref, q1_ref, y_ref, d0_ref, d1_ref, w_ref, b2_ref, bh_ref, o_ref):
        dinv = _dinv(d0_ref, d1_ref)
        h = jnp.maximum(b2_ref[...] + dinv * (q0_ref[...] + q1_ref[...] + y_ref[...]), 0.0)
        o_ref[...] = jnp.dot(h, w_ref[...],
                             preferred_element_type=jnp.float32) + bh_ref[...]

    return pl.pallas_call(
        body,
        grid=(_GRID,),
        in_specs=[
            pl.BlockSpec((_BLK, 128), lambda i: (i, 0)),
            pl.BlockSpec((_BLK, 128), lambda i: (i, 0)),
            pl.BlockSpec((_BLK, 128), lambda i: (i, 0)),
            pl.BlockSpec((_BLK, 128), lambda i: (i, 0)),
            pl.BlockSpec((_BLK, 128), lambda i: (i, 0)),
            pl.BlockSpec((128, 16), lambda i: (0, 0)),
            pl.BlockSpec((1, 128), lambda i: (0, 0)),
            pl.BlockSpec((1, 16), lambda i: (0, 0)),
        ],
        out_specs=pl.BlockSpec((_BLK, 16), lambda i: (i, 0)),
        out_shape=jax.ShapeDtypeStruct((_NP, 16), jnp.float32),
    )(q0, q1, y2, d0, d1, wh, b2r, bhr)


def kernel(obs, edge_index, W1, b1, W2, b2, Wm, bm, Wl, bl):
    f32 = jnp.float32
    obs_p = jnp.zeros((_NP, obs.shape[1]), f32).at[:_N].set(obs)
    src = edge_index[0]
    dst = edge_index[1]
    padn = _EP - _E
    src_p = jnp.concatenate(
        [src, jnp.zeros((padn,), jnp.int32)]).reshape(_NW, _KD, _CH)
    dst_p = jnp.concatenate(
        [dst, jnp.full((padn,), _N, jnp.int32)]).reshape(_NW, _KD, _CH)
    onehot = jnp.zeros((_CH, 128), f32).at[:, 0].set(1.0)
    zeros128 = jnp.zeros((_CH, 128), f32)

    degp = _sc_degree(dst_p, onehot, zeros128).reshape(2, _NP, 128)
    d0, d1 = degp[0], degp[1]

    src_f = src_p.reshape(_EP // _CH, _CH)
    dst_f = dst_p.reshape(_EP // _CH, _CH)
    y1 = _mm_scale(obs_p, W1, d0, d1)
    p = _sc_agg(y1, src_f, dst_f, zeros128).reshape(2, _NP, 128)
    y2 = _layer2(p[0], p[1], y1, d0, d1, W2, b1.reshape(1, -1))
    q = _sc_agg(y2, src_f, dst_f, zeros128).reshape(2, _NP, 128)

    wh = jnp.concatenate([Wm, Wl], axis=1)
    bh = jnp.concatenate([bm, bl]).reshape(1, -1)
    heads = _final(q[0], q[1], y2, d0, d1, wh, b2.reshape(1, -1), bh)
    return heads[:_N, :8], heads[:_N, 8:]


# speedup vs baseline: 20.2329x; 2.1870x over previous
import functools

import jax
import jax.numpy as jnp
from jax import lax
from jax.experimental import pallas as pl
from jax.experimental.pallas import tpu as pltpu
from jax.experimental.pallas import tpu_sc as plsc

_N = 10000
_E = 320000
_NP = 10240
_CH = 128
_NW = 32
_KD = 80
_GI = 16
_KD0 = 80
_KD1 = 80
_EP = _NW * _KD * _CH
_RPS = _NP // 16
_BLK = 512
_GRID = _NP // _BLK

_MESH = plsc.VectorSubcoreMesh(core_axis_name="c", subcore_axis_name="s")


@functools.partial(
    pl.kernel,
    out_type=jax.ShapeDtypeStruct((2, 16, _RPS, 128), jnp.float32),
    mesh=_MESH,
    scratch_types=[
        pltpu.VMEM((_KD, _CH), jnp.int32),
        pltpu.VMEM((_CH, 128), jnp.float32),
        pltpu.VMEM_SHARED((_NP, 128), jnp.float32),
    ],
)
def _sc_degree(dst_hbm, onehot_hbm, zeros_hbm, out_hbm, dst_v, buf_v, acc_sh):
    c = lax.axis_index("c")
    s = lax.axis_index("s")
    wid = s * 2 + c
    r0 = s * _RPS
    pltpu.sync_copy(zeros_hbm, buf_v)
    for i in range(_RPS // _CH):
        pltpu.sync_copy(buf_v, acc_sh.at[pl.ds(r0 + i * _CH, _CH)])
    pltpu.sync_copy(onehot_hbm, buf_v)
    pltpu.sync_copy(dst_hbm.at[wid], dst_v)
    plsc.subcore_barrier()

    def body(j, carry):
        pltpu.sync_copy(buf_v, acc_sh.at[dst_v.at[j]], add=True)
        return carry

    lax.fori_loop(0, _KD, body, 0)
    plsc.subcore_barrier()
    for i in range(_RPS // _CH):
        pltpu.sync_copy(acc_sh.at[pl.ds(r0 + i * _CH, _CH)], buf_v)
        pltpu.sync_copy(buf_v, out_hbm.at[c, s, pl.ds(i * _CH, _CH)])


@functools.partial(
    pl.kernel,
    out_type=jax.ShapeDtypeStruct((2, 16, _RPS, 128), jnp.float32),
    mesh=_MESH,
    scratch_types=[
        pltpu.VMEM((_GI, _CH), jnp.int32),
        pltpu.VMEM((_GI, _CH), jnp.int32),
        pltpu.VMEM((_CH, 128), jnp.float32),
        pltpu.VMEM((_CH, 128), jnp.float32),
        pltpu.VMEM_SHARED((_NP, 128), jnp.float32),
        pltpu.SemaphoreType.DMA,
        pltpu.SemaphoreType.DMA,
    ],
)
def _sc_agg(y_hbm, src_hbm, dst_hbm, zeros_hbm, out_hbm,
            src_v, dst_v, m0, m1, acc_sh, sem0, sem1):
    c = lax.axis_index("c")
    s = lax.axis_index("s")
    r0 = s * _RPS
    pltpu.sync_copy(zeros_hbm, m0)
    for i in range(_RPS // _CH):
        pltpu.sync_copy(m0, acc_sh.at[pl.ds(r0 + i * _CH, _CH)])
    plsc.subcore_barrier()

    def do_edges(base, nkd):
        for g in range(nkd // _GI):
            pltpu.sync_copy(src_hbm.at[pl.ds(base + g * _GI, _GI)], src_v)
            pltpu.sync_copy(dst_hbm.at[pl.ds(base + g * _GI, _GI)], dst_v)
            pltpu.async_copy(y_hbm.at[src_v.at[0]], m0, sem0)

            def body(jj, carry):
                j = jj * 2
                pltpu.make_async_copy(y_hbm.at[src_v.at[j]], m0, sem0).wait()
                pltpu.async_copy(y_hbm.at[src_v.at[j + 1]], m1, sem1)
                pltpu.sync_copy(m0, acc_sh.at[dst_v.at[j]], add=True)
                pltpu.make_async_copy(y_hbm.at[src_v.at[j + 1]], m1, sem1).wait()

                @pl.when(jj + 1 < _GI // 2)
                def _():
                    pltpu.async_copy(y_hbm.at[src_v.at[j + 2]], m0, sem0)

                pltpu.sync_copy(m1, acc_sh.at[dst_v.at[j + 1]], add=True)
                return carry

            lax.fori_loop(0, _GI // 2, body, 0)

    @pl.when(c == 0)
    def _():
        do_edges(s * _KD0, _KD0)

    @pl.when(c == 1)
    def _():
        do_edges(16 * _KD0 + s * _KD1, _KD1)

    plsc.subcore_barrier()
    for i in range(_RPS // _CH):
        pltpu.sync_copy(acc_sh.at[pl.ds(r0 + i * _CH, _CH)], m0)
        pltpu.sync_copy(m0, out_hbm.at[c, s, pl.ds(i * _CH, _CH)])


def _dinv(d0_ref, d1_ref):
    return lax.rsqrt(1.0 + d0_ref[:, 0:1] + d1_ref[:, 0:1])


def _mm_scale(x, w, d0, d1):
    def body(x_ref, w_ref, d0_ref, d1_ref, o_ref):
        dinv = _dinv(d0_ref, d1_ref)
        o_ref[...] = jnp.dot(x_ref[...], w_ref[...],
                             preferred_element_type=jnp.float32) * dinv

    return pl.pallas_call(
        body,
        grid=(_GRID,),
        in_specs=[
            pl.BlockSpec((_BLK, 128), lambda i: (i, 0)),
            pl.BlockSpec((128, 128), lambda i: (0, 0)),
            pl.BlockSpec((_BLK, 128), lambda i: (i, 0)),
            pl.BlockSpec((_BLK, 128), lambda i: (i, 0)),
        ],
        out_specs=pl.BlockSpec((_BLK, 128), lambda i: (i, 0)),
        out_shape=jax.ShapeDtypeStruct((_NP, 128), jnp.float32),
    )(x, w, d0, d1)


def _layer2(p0, p1, y1, d0, d1, w2, b1r):
    def body(p0_ref, p1_ref, y_ref, d0_ref, d1_ref, w_ref, b_ref, o_ref):
        dinv = _dinv(d0_ref, d1_ref)
        h = jnp.maximum(b_ref[...] + dinv * (p0_ref[...] + p1_ref[...] + y_ref[...]), 0.0)
        o_ref[...] = jnp.dot(h, w_ref[...],
                             preferred_element_type=jnp.float32) * dinv

    return pl.pallas_call(
        body,
        grid=(_GRID,),
        in_specs=[
            pl.BlockSpec((_BLK, 128), lambda i: (i, 0)),
            pl.BlockSpec((_BLK, 128), lambda i: (i, 0)),
            pl.BlockSpec((_BLK, 128), lambda i: (i, 0)),
            pl.BlockSpec((_BLK, 128), lambda i: (i, 0)),
            pl.BlockSpec((_BLK, 128), lambda i: (i, 0)),
            pl.BlockSpec((128, 128), lambda i: (0, 0)),
            pl.BlockSpec((1, 128), lambda i: (0, 0)),
        ],
        out_specs=pl.BlockSpec((_BLK, 128), lambda i: (i, 0)),
        out_shape=jax.ShapeDtypeStruct((_NP, 128), jnp.float32),
    )(p0, p1, y1, d0, d1, w2, b1r)


def _final(q0, q1, y2, d0, d1, wh, b2r, bhr):
    def body(q0_ref, q1_ref, y_ref, d0_ref, d1_ref, w_ref, b2_ref, bh_ref, o_ref):
        dinv = _dinv(d0_ref, d1_ref)
        h = jnp.maximum(b2_ref[...] + dinv * (q0_ref[...] + q1_ref[...] + y_ref[...]), 0.0)
        o_ref[...] = jnp.dot(h, w_ref[...],
                             preferred_element_type=jnp.float32) + bh_ref[...]

    return pl.pallas_call(
        body,
        grid=(_GRID,),
        in_specs=[
            pl.BlockSpec((_BLK, 128), lambda i: (i, 0)),
            pl.BlockSpec((_BLK, 128), lambda i: (i, 0)),
            pl.BlockSpec((_BLK, 128), lambda i: (i, 0)),
            pl.BlockSpec((_BLK, 128), lambda i: (i, 0)),
            pl.BlockSpec((_BLK, 128), lambda i: (i, 0)),
            pl.BlockSpec((128, 16), lambda i: (0, 0)),
            pl.BlockSpec((1, 128), lambda i: (0, 0)),
            pl.BlockSpec((1, 16), lambda i: (0, 0)),
        ],
        out_specs=pl.BlockSpec((_BLK, 16), lambda i: (i, 0)),
        out_shape=jax.ShapeDtypeStruct((_NP, 16), jnp.float32),
    )(q0, q1, y2, d0, d1, wh, b2r, bhr)


def kernel(obs, edge_index, W1, b1, W2, b2, Wm, bm, Wl, bl):
    f32 = jnp.float32
    obs_p = jnp.zeros((_NP, obs.shape[1]), f32).at[:_N].set(obs)
    src = edge_index[0]
    dst = edge_index[1]
    padn = _EP - _E
    pad_i = jnp.arange(padn, dtype=jnp.int32)
    src_p = jnp.concatenate(
        [src, pad_i % _N]).reshape(_NW, _KD, _CH)
    dst_p = jnp.concatenate(
        [dst, _N + pad_i % (_NP - _N)]).reshape(_NW, _KD, _CH)
    onehot = jnp.zeros((_CH, 128), f32).at[:, 0].set(1.0)
    zeros128 = jnp.zeros((_CH, 128), f32)

    degp = _sc_degree(dst_p, onehot, zeros128).reshape(2, _NP, 128)
    d0, d1 = degp[0], degp[1]

    src_f = src_p.reshape(_EP // _CH, _CH)
    dst_f = dst_p.reshape(_EP // _CH, _CH)
    y1 = _mm_scale(obs_p, W1, d0, d1)
    p = _sc_agg(y1, src_f, dst_f, zeros128).reshape(2, _NP, 128)
    y2 = _layer2(p[0], p[1], y1, d0, d1, W2, b1.reshape(1, -1))
    q = _sc_agg(y2, src_f, dst_f, zeros128).reshape(2, _NP, 128)

    wh = jnp.concatenate([Wm, Wl], axis=1)
    bh = jnp.concatenate([bm, bl]).reshape(1, -1)
    heads = _final(q[0], q[1], y2, d0, d1, wh, b2.reshape(1, -1), bh)
    return heads[:_N, :8], heads[:_N, 8:]


# speedup vs baseline: 21.1023x; 1.0430x over previous
import functools

import jax
import jax.numpy as jnp
from jax import lax
from jax.experimental import pallas as pl
from jax.experimental.pallas import tpu as pltpu
from jax.experimental.pallas import tpu_sc as plsc

_N = 10000
_E = 320000
_NP = 10240
_CH = 128
_NW = 32
_KD = 80
_GI = 16
_KD0 = 80
_KD1 = 80
_EP = _NW * _KD * _CH
_RPS = _NP // 16
_BLK = 1000
_GRID = _N // _BLK

_MESH = plsc.VectorSubcoreMesh(core_axis_name="c", subcore_axis_name="s")


@functools.partial(
    pl.kernel,
    out_type=jax.ShapeDtypeStruct((2, 16, _RPS, 128), jnp.float32),
    mesh=_MESH,
    scratch_types=[
        pltpu.VMEM((_KD, _CH), jnp.int32),
        pltpu.VMEM((_CH, 128), jnp.float32),
        pltpu.VMEM_SHARED((_NP, 128), jnp.float32),
        pltpu.SemaphoreType.DMA,
    ],
)
def _sc_degree(dst_hbm, onehot_hbm, zeros_hbm, out_hbm, dst_v, buf_v, acc_sh, dsem):
    c = lax.axis_index("c")
    s = lax.axis_index("s")
    wid = s * 2 + c
    r0 = s * _RPS
    pltpu.sync_copy(zeros_hbm, buf_v)
    for i in range(_RPS // _CH):
        pltpu.sync_copy(buf_v, acc_sh.at[pl.ds(r0 + i * _CH, _CH)])
    pltpu.sync_copy(onehot_hbm, buf_v)
    pltpu.sync_copy(dst_hbm.at[wid], dst_v)
    plsc.subcore_barrier()

    def body(jj, carry):
        j0 = jj * 8
        for k in range(8):
            pltpu.async_copy(buf_v, acc_sh.at[dst_v.at[j0 + k]], dsem, add=True)
        for k in range(8):
            pltpu.make_async_copy(buf_v, acc_sh.at[dst_v.at[j0 + k]], dsem).wait()
        return carry

    lax.fori_loop(0, _KD // 8, body, 0)
    plsc.subcore_barrier()
    for i in range(_RPS // _CH):
        pltpu.sync_copy(acc_sh.at[pl.ds(r0 + i * _CH, _CH)], buf_v)
        pltpu.sync_copy(buf_v, out_hbm.at[c, s, pl.ds(i * _CH, _CH)])


@functools.partial(
    pl.kernel,
    out_type=jax.ShapeDtypeStruct((2, 16, _RPS, 128), jnp.float32),
    mesh=_MESH,
    scratch_types=[
        pltpu.VMEM((_GI, _CH), jnp.int32),
        pltpu.VMEM((_GI, _CH), jnp.int32),
        pltpu.VMEM((_CH, 128), jnp.float32),
        pltpu.VMEM((_CH, 128), jnp.float32),
        pltpu.VMEM_SHARED((_NP, 128), jnp.float32),
        pltpu.SemaphoreType.DMA,
        pltpu.SemaphoreType.DMA,
        pltpu.SemaphoreType.DMA,
        pltpu.SemaphoreType.DMA,
    ],
)
def _sc_agg(y_hbm, src_hbm, dst_hbm, zeros_hbm, out_hbm,
            src_v, dst_v, m0, m1, acc_sh, sg0, sg1, ss0, ss1):
    c = lax.axis_index("c")
    s = lax.axis_index("s")
    r0 = s * _RPS
    pltpu.sync_copy(zeros_hbm, m0)
    for i in range(_RPS // _CH):
        pltpu.sync_copy(m0, acc_sh.at[pl.ds(r0 + i * _CH, _CH)])
    plsc.subcore_barrier()

    def do_edges(base, nkd):
        for g in range(nkd // _GI):
            pltpu.sync_copy(src_hbm.at[pl.ds(base + g * _GI, _GI)], src_v)
            pltpu.sync_copy(dst_hbm.at[pl.ds(base + g * _GI, _GI)], dst_v)
            pltpu.async_copy(y_hbm.at[src_v.at[0]], m0, sg0)

            def body(jj, carry):
                j = jj * 2
                pltpu.make_async_copy(y_hbm.at[src_v.at[j]], m0, sg0).wait()

                @pl.when(jj > 0)
                def _():
                    pltpu.make_async_copy(m1, acc_sh.at[dst_v.at[j - 1]], ss1).wait()

                pltpu.async_copy(y_hbm.at[src_v.at[j + 1]], m1, sg1)
                pltpu.async_copy(m0, acc_sh.at[dst_v.at[j]], ss0, add=True)
                pltpu.make_async_copy(y_hbm.at[src_v.at[j + 1]], m1, sg1).wait()

                @pl.when(jj + 1 < _GI // 2)
                def _():
                    pltpu.make_async_copy(m0, acc_sh.at[dst_v.at[j]], ss0).wait()
                    pltpu.async_copy(y_hbm.at[src_v.at[j + 2]], m0, sg0)

                pltpu.async_copy(m1, acc_sh.at[dst_v.at[j + 1]], ss1, add=True)
                return carry

            lax.fori_loop(0, _GI // 2, body, 0)
            pltpu.make_async_copy(m0, acc_sh.at[dst_v.at[_GI - 2]], ss0).wait()
            pltpu.make_async_copy(m1, acc_sh.at[dst_v.at[_GI - 1]], ss1).wait()

    @pl.when(c == 0)
    def _():
        do_edges(s * _KD0, _KD0)

    @pl.when(c == 1)
    def _():
        do_edges(16 * _KD0 + s * _KD1, _KD1)

    plsc.subcore_barrier()
    for i in range(_RPS // _CH):
        pltpu.sync_copy(acc_sh.at[pl.ds(r0 + i * _CH, _CH)], m0)
        pltpu.sync_copy(m0, out_hbm.at[c, s, pl.ds(i * _CH, _CH)])


def _dinv(d0_ref, d1_ref):
    return lax.rsqrt(1.0 + d0_ref[:, 0:1] + d1_ref[:, 0:1])


def _mm_scale(x, w, d0, d1):
    def body(x_ref, w_ref, d0_ref, d1_ref, o_ref):
        dinv = _dinv(d0_ref, d1_ref)
        o_ref[...] = jnp.dot(x_ref[...], w_ref[...],
                             preferred_element_type=jnp.float32) * dinv

    return pl.pallas_call(
        body,
        grid=(_GRID,),
        in_specs=[
            pl.BlockSpec((_BLK, 128), lambda i: (i, 0)),
            pl.BlockSpec((128, 128), lambda i: (0, 0)),
            pl.BlockSpec((_BLK, 128), lambda i: (i, 0)),
            pl.BlockSpec((_BLK, 128), lambda i: (i, 0)),
        ],
        out_specs=pl.BlockSpec((_BLK, 128), lambda i: (i, 0)),
        out_shape=jax.ShapeDtypeStruct((_N, 128), jnp.float32),
    )(x, w, d0, d1)


def _layer2(p0, p1, y1, d0, d1, w2, b1r):
    def body(p0_ref, p1_ref, y_ref, d0_ref, d1_ref, w_ref, b_ref, o_ref):
        dinv = _dinv(d0_ref, d1_ref)
        h = jnp.maximum(b_ref[...] + dinv * (p0_ref[...] + p1_ref[...] + y_ref[...]), 0.0)
        o_ref[...] = jnp.dot(h, w_ref[...],
                             preferred_element_type=jnp.float32) * dinv

    return pl.pallas_call(
        body,
        grid=(_GRID,),
        in_specs=[
            pl.BlockSpec((_BLK, 128), lambda i: (i, 0)),
            pl.BlockSpec((_BLK, 128), lambda i: (i, 0)),
            pl.BlockSpec((_BLK, 128), lambda i: (i, 0)),
            pl.BlockSpec((_BLK, 128), lambda i: (i, 0)),
            pl.BlockSpec((_BLK, 128), lambda i: (i, 0)),
            pl.BlockSpec((128, 128), lambda i: (0, 0)),
            pl.BlockSpec((1, 128), lambda i: (0, 0)),
        ],
        out_specs=pl.BlockSpec((_BLK, 128), lambda i: (i, 0)),
        out_shape=jax.ShapeDtypeStruct((_N, 128), jnp.float32),
    )(p0, p1, y1, d0, d1, w2, b1r)


def _final(q0, q1, y2, d0, d1, wh, b2r, bhr):
    def body(q0_ref, q1_ref, y_ref, d0_ref, d1_ref, w_ref, b2_ref, bh_ref, o_ref):
        dinv = _dinv(d0_ref, d1_ref)
        h = jnp.maximum(b2_ref[...] + dinv * (q0_ref[...] + q1_ref[...] + y_ref[...]), 0.0)
        o_ref[...] = jnp.dot(h, w_ref[...],
                             preferred_element_type=jnp.float32) + bh_ref[...]

    return pl.pallas_call(
        body,
        grid=(_GRID,),
        in_specs=[
            pl.BlockSpec((_BLK, 128), lambda i: (i, 0)),
            pl.BlockSpec((_BLK, 128), lambda i: (i, 0)),
            pl.BlockSpec((_BLK, 128), lambda i: (i, 0)),
            pl.BlockSpec((_BLK, 128), lambda i: (i, 0)),
            pl.BlockSpec((_BLK, 128), lambda i: (i, 0)),
            pl.BlockSpec((128, 16), lambda i: (0, 0)),
            pl.BlockSpec((1, 128), lambda i: (0, 0)),
            pl.BlockSpec((1, 16), lambda i: (0, 0)),
        ],
        out_specs=pl.BlockSpec((_BLK, 16), lambda i: (i, 0)),
        out_shape=jax.ShapeDtypeStruct((_N, 16), jnp.float32),
    )(q0, q1, y2, d0, d1, wh, b2r, bhr)


def kernel(obs, edge_index, W1, b1, W2, b2, Wm, bm, Wl, bl):
    f32 = jnp.float32
    src = edge_index[0]
    dst = edge_index[1]
    padn = _EP - _E
    pad_i = jnp.arange(padn, dtype=jnp.int32)
    src_p = jnp.concatenate(
        [src, pad_i % _N]).reshape(_NW, _KD, _CH)
    dst_p = jnp.concatenate(
        [dst, _N + pad_i % (_NP - _N)]).reshape(_NW, _KD, _CH)
    onehot = jnp.zeros((_CH, 128), f32).at[:, 0].set(1.0)
    zeros128 = jnp.zeros((_CH, 128), f32)

    degp = _sc_degree(dst_p, onehot, zeros128).reshape(2, _NP, 128)
    d0, d1 = degp[0], degp[1]

    src_f = src_p.reshape(_EP // _CH, _CH)
    dst_f = dst_p.reshape(_EP // _CH, _CH)
    y1 = _mm_scale(obs, W1, d0, d1)
    p = _sc_agg(y1, src_f, dst_f, zeros128).reshape(2, _NP, 128)
    y2 = _layer2(p[0], p[1], y1, d0, d1, W2, b1.reshape(1, -1))
    q = _sc_agg(y2, src_f, dst_f, zeros128).reshape(2, _NP, 128)

    wh = jnp.concatenate([Wm, Wl], axis=1)
    bh = jnp.concatenate([bm, bl]).reshape(1, -1)
    heads = _final(q[0], q[1], y2, d0, d1, wh, b2.reshape(1, -1), bh)
    return heads[:, :8], heads[:, 8:]


# speedup vs baseline: 23.0794x; 1.0937x over previous
import functools

import jax
import jax.numpy as jnp
from jax import lax
from jax.experimental import pallas as pl
from jax.experimental.pallas import tpu as pltpu
from jax.experimental.pallas import tpu_sc as plsc

_N = 10000
_E = 320000
_NP = 10240
_CH = 128
_NW = 32
_KD = 80
_GI = 16
_CHA = 64
_GIA = 32
_KD0 = 160
_KD1 = 160
_EP = _NW * _KD * _CH
_RPS = _NP // 16
_BLK = 1000
_GRID = _N // _BLK

_MESH = plsc.VectorSubcoreMesh(core_axis_name="c", subcore_axis_name="s")


@functools.partial(
    pl.kernel,
    out_type=jax.ShapeDtypeStruct((2, 16, _RPS, 128), jnp.float32),
    mesh=_MESH,
    scratch_types=[
        pltpu.VMEM((_KD, _CH), jnp.int32),
        pltpu.VMEM((_CH, 128), jnp.float32),
        pltpu.VMEM_SHARED((_NP, 128), jnp.float32),
        pltpu.SemaphoreType.DMA,
    ],
)
def _sc_degree(dst_hbm, onehot_hbm, zeros_hbm, out_hbm, dst_v, buf_v, acc_sh, dsem):
    c = lax.axis_index("c")
    s = lax.axis_index("s")
    wid = s * 2 + c
    r0 = s * _RPS
    pltpu.sync_copy(zeros_hbm, buf_v)
    for i in range(_RPS // _CH):
        pltpu.sync_copy(buf_v, acc_sh.at[pl.ds(r0 + i * _CH, _CH)])
    pltpu.sync_copy(onehot_hbm, buf_v)
    pltpu.sync_copy(dst_hbm.at[wid], dst_v)
    plsc.subcore_barrier()

    def body(jj, carry):
        j0 = jj * 8
        for k in range(8):
            pltpu.async_copy(buf_v, acc_sh.at[dst_v.at[j0 + k]], dsem, add=True)
        for k in range(8):
            pltpu.make_async_copy(buf_v, acc_sh.at[dst_v.at[j0 + k]], dsem).wait()
        return carry

    lax.fori_loop(0, _KD // 8, body, 0)
    plsc.subcore_barrier()
    for i in range(_RPS // _CH):
        pltpu.sync_copy(acc_sh.at[pl.ds(r0 + i * _CH, _CH)], buf_v)
        pltpu.sync_copy(buf_v, out_hbm.at[c, s, pl.ds(i * _CH, _CH)])


@functools.partial(
    pl.kernel,
    out_type=jax.ShapeDtypeStruct((2, 16, _RPS, 128), jnp.float32),
    mesh=_MESH,
    scratch_types=[
        pltpu.VMEM((_GIA, _CHA), jnp.int32),
        pltpu.VMEM((_GIA, _CHA), jnp.int32),
        pltpu.VMEM((_CHA, 128), jnp.float32),
        pltpu.VMEM((_CHA, 128), jnp.float32),
        pltpu.VMEM((_CHA, 128), jnp.float32),
        pltpu.VMEM((_CHA, 128), jnp.float32),
        pltpu.VMEM_SHARED((_NP, 128), jnp.float32),
        pltpu.SemaphoreType.DMA,
        pltpu.SemaphoreType.DMA,
        pltpu.SemaphoreType.DMA,
        pltpu.SemaphoreType.DMA,
        pltpu.SemaphoreType.DMA,
        pltpu.SemaphoreType.DMA,
        pltpu.SemaphoreType.DMA,
        pltpu.SemaphoreType.DMA,
    ],
)
def _sc_agg(y_hbm, src_hbm, dst_hbm, zeros_hbm, out_hbm,
            src_v, dst_v, b0, b1, b2, b3, acc_sh,
            gs0, gs1, gs2, gs3, ss0, ss1, ss2, ss3):
    c = lax.axis_index("c")
    s = lax.axis_index("s")
    r0 = s * _RPS
    bufs = (b0, b1, b2, b3)
    gsem = (gs0, gs1, gs2, gs3)
    ssem = (ss0, ss1, ss2, ss3)
    pltpu.sync_copy(zeros_hbm, b0)
    pltpu.sync_copy(zeros_hbm, b1)
    for i in range(_RPS // (2 * _CHA)):
        pltpu.sync_copy(b0, acc_sh.at[pl.ds(r0 + (2 * i) * _CHA, _CHA)])
        pltpu.sync_copy(b1, acc_sh.at[pl.ds(r0 + (2 * i + 1) * _CHA, _CHA)])
    plsc.subcore_barrier()

    def do_edges(base, nkd):
        for g in range(nkd // _GIA):
            pltpu.sync_copy(src_hbm.at[pl.ds(base + g * _GIA, _GIA)], src_v)
            pltpu.sync_copy(dst_hbm.at[pl.ds(base + g * _GIA, _GIA)], dst_v)
            for b in range(3):
                pltpu.async_copy(y_hbm.at[src_v.at[b]], bufs[b], gsem[b])

            def body(jj, carry):
                j = jj * 4
                for b in range(4):
                    k = j + b
                    pf = k + 3
                    pb = (b + 3) % 4
                    pltpu.make_async_copy(
                        y_hbm.at[src_v.at[k]], bufs[b], gsem[b]).wait()
                    pltpu.async_copy(
                        bufs[b], acc_sh.at[dst_v.at[k]], ssem[b], add=True)
                    if b == 0:
                        @pl.when(jj > 0)
                        def _(pb=pb, k=k):
                            pltpu.make_async_copy(
                                bufs[pb], acc_sh.at[dst_v.at[k - 1]],
                                ssem[pb]).wait()

                        pltpu.async_copy(
                            y_hbm.at[src_v.at[pf]], bufs[pb], gsem[pb])
                    else:
                        @pl.when(jj < _GIA // 4 - 1)
                        def _(pb=pb, k=k, pf=pf):
                            pltpu.make_async_copy(
                                bufs[pb], acc_sh.at[dst_v.at[k - 1]],
                                ssem[pb]).wait()
                            pltpu.async_copy(
                                y_hbm.at[src_v.at[pf]], bufs[pb], gsem[pb])
                return carry

            lax.fori_loop(0, _GIA // 4, body, 0)
            for b in range(4):
                pltpu.make_async_copy(
                    bufs[b], acc_sh.at[dst_v.at[_GIA - 4 + b]], ssem[b]).wait()

    @pl.when(c == 0)
    def _():
        do_edges(s * _KD0, _KD0)

    @pl.when(c == 1)
    def _():
        do_edges(16 * _KD0 + s * _KD1, _KD1)

    plsc.subcore_barrier()
    for i in range(_RPS // (2 * _CHA)):
        pltpu.sync_copy(acc_sh.at[pl.ds(r0 + (2 * i) * _CHA, _CHA)], b0)
        pltpu.sync_copy(b0, out_hbm.at[c, s, pl.ds((2 * i) * _CHA, _CHA)])
        pltpu.sync_copy(acc_sh.at[pl.ds(r0 + (2 * i + 1) * _CHA, _CHA)], b1)
        pltpu.sync_copy(b1, out_hbm.at[c, s, pl.ds((2 * i + 1) * _CHA, _CHA)])


def _dinv(d0_ref, d1_ref):
    return lax.rsqrt(1.0 + d0_ref[:, 0:1] + d1_ref[:, 0:1])


def _mm_scale(x, w, d0, d1):
    def body(x_ref, w_ref, d0_ref, d1_ref, o_ref):
        dinv = _dinv(d0_ref, d1_ref)
        o_ref[...] = jnp.dot(x_ref[...], w_ref[...],
                             preferred_element_type=jnp.float32) * dinv

    return pl.pallas_call(
        body,
        grid=(_GRID,),
        in_specs=[
            pl.BlockSpec((_BLK, 128), lambda i: (i, 0)),
            pl.BlockSpec((128, 128), lambda i: (0, 0)),
            pl.BlockSpec((_BLK, 128), lambda i: (i, 0)),
            pl.BlockSpec((_BLK, 128), lambda i: (i, 0)),
        ],
        out_specs=pl.BlockSpec((_BLK, 128), lambda i: (i, 0)),
        out_shape=jax.ShapeDtypeStruct((_N, 128), jnp.float32),
    )(x, w, d0, d1)


def _layer2(p0, p1, y1, d0, d1, w2, b1r):
    def body(p0_ref, p1_ref, y_ref, d0_ref, d1_ref, w_ref, b_ref, o_ref):
        dinv = _dinv(d0_ref, d1_ref)
        h = jnp.maximum(b_ref[...] + dinv * (p0_ref[...] + p1_ref[...] + y_ref[...]), 0.0)
        o_ref[...] = jnp.dot(h, w_ref[...],
                             preferred_element_type=jnp.float32) * dinv

    return pl.pallas_call(
        body,
        grid=(_GRID,),
        in_specs=[
            pl.BlockSpec((_BLK, 128), lambda i: (i, 0)),
            pl.BlockSpec((_BLK, 128), lambda i: (i, 0)),
            pl.BlockSpec((_BLK, 128), lambda i: (i, 0)),
            pl.BlockSpec((_BLK, 128), lambda i: (i, 0)),
            pl.BlockSpec((_BLK, 128), lambda i: (i, 0)),
            pl.BlockSpec((128, 128), lambda i: (0, 0)),
            pl.BlockSpec((1, 128), lambda i: (0, 0)),
        ],
        out_specs=pl.BlockSpec((_BLK, 128), lambda i: (i, 0)),
        out_shape=jax.ShapeDtypeStruct((_N, 128), jnp.float32),
    )(p0, p1, y1, d0, d1, w2, b1r)


def _final(q0, q1, y2, d0, d1, wh, b2r, bhr):
    def body(q0_ref, q1_ref, y_ref, d0_ref, d1_ref, w_ref, b2_ref, bh_ref, o_ref):
        dinv = _dinv(d0_ref, d1_ref)
        h = jnp.maximum(b2_ref[...] + dinv * (q0_ref[...] + q1_ref[...] + y_ref[...]), 0.0)
        o_ref[...] = jnp.dot(h, w_ref[...],
                             preferred_element_type=jnp.float32) + bh_ref[...]

    return pl.pallas_call(
        body,
        grid=(_GRID,),
        in_specs=[
            pl.BlockSpec((_BLK, 128), lambda i: (i, 0)),
            pl.BlockSpec((_BLK, 128), lambda i: (i, 0)),
            pl.BlockSpec((_BLK, 128), lambda i: (i, 0)),
            pl.BlockSpec((_BLK, 128), lambda i: (i, 0)),
            pl.BlockSpec((_BLK, 128), lambda i: (i, 0)),
            pl.BlockSpec((128, 16), lambda i: (0, 0)),
            pl.BlockSpec((1, 128), lambda i: (0, 0)),
            pl.BlockSpec((1, 16), lambda i: (0, 0)),
        ],
        out_specs=pl.BlockSpec((_BLK, 16), lambda i: (i, 0)),
        out_shape=jax.ShapeDtypeStruct((_N, 16), jnp.float32),
    )(q0, q1, y2, d0, d1, wh, b2r, bhr)


def kernel(obs, edge_index, W1, b1, W2, b2, Wm, bm, Wl, bl):
    f32 = jnp.float32
    src = edge_index[0]
    dst = edge_index[1]
    padn = _EP - _E
    pad_i = jnp.arange(padn, dtype=jnp.int32)
    src_p = jnp.concatenate(
        [src, pad_i % _N]).reshape(_NW, _KD, _CH)
    dst_p = jnp.concatenate(
        [dst, _N + pad_i % (_NP - _N)]).reshape(_NW, _KD, _CH)
    onehot = jnp.zeros((_CH, 128), f32).at[:, 0].set(1.0)
    zeros128 = jnp.zeros((_CH, 128), f32)

    degp = _sc_degree(dst_p, onehot, zeros128).reshape(2, _NP, 128)
    d0, d1 = degp[0], degp[1]

    src_f = src_p.reshape(_EP // _CHA, _CHA)
    dst_f = dst_p.reshape(_EP // _CHA, _CHA)
    zeros64 = jnp.zeros((_CHA, 128), f32)
    y1 = _mm_scale(obs, W1, d0, d1)
    p = _sc_agg(y1, src_f, dst_f, zeros64).reshape(2, _NP, 128)
    y2 = _layer2(p[0], p[1], y1, d0, d1, W2, b1.reshape(1, -1))
    q = _sc_agg(y2, src_f, dst_f, zeros64).reshape(2, _NP, 128)

    wh = jnp.concatenate([Wm, Wl], axis=1)
    bh = jnp.concatenate([bm, bl]).reshape(1, -1)
    heads = _final(q[0], q[1], y2, d0, d1, wh, b2.reshape(1, -1), bh)
    return heads[:, :8], heads[:, 8:]
